# baseline (speedup 1.0000x reference)
import numpy as np
import jax
import jax.numpy as jnp
from jax import lax
from jax.experimental import pallas as pl
from jax.experimental.pallas import tpu as pltpu
from jax.experimental.pallas import tpu_sc as plsc

B = 16384
L = 200
E = 64
NC = 2
NS = 16
NW = NC * NS
ROWS_PER_W = B // NW
IDXW = 128
RPB = 16
SPB = (RPB * L) // IDXW
NBLK = ROWS_PER_W // RPB
STREAMS_PER_W = NBLK * SPB
NRING = 5
OCHUNK = 64
LANES = 16


def _row_segments():
    segs = []
    for k in range(RPB):
        lo, hi = L * k, L * k + L
        row = []
        s = lo // IDXW
        while lo < hi:
            take = min(hi, (s + 1) * IDXW) - lo
            row.append((s, lo - s * IDXW, take))
            lo += take
            s += 1
        segs.append(row)
    return segs


SEGS = _row_segments()
_first = {}
_last = {}
for _k, _row in enumerate(SEGS):
    for _s, _, _ in _row:
        _first.setdefault(_s, _k)
        _last[_s] = _k
NEW_STREAMS = [[s for s in range(SPB) if _first[s] == k] for k in range(RPB)]
DEAD_STREAMS = [[s for s in range(SPB) if _last[s] == k] for k in range(RPB)]


def _awe_body(x_hbm, table_hbm, out_hbm, idx_v, gath_v, outb_v,
              sem0, sem1, sem2, sem3, sem4, semi):
    wid = lax.axis_index("s") * NC + lax.axis_index("c")
    sems = (sem0, sem1, sem2, sem3, sem4)

    def stage_idx(m_next, ibuf):
        pltpu.async_copy(
            x_hbm.at[pl.ds(wid * STREAMS_PER_W + m_next * SPB, SPB)],
            idx_v.at[ibuf], semi)
        pltpu.make_async_copy(x_hbm.at[pl.ds(0, SPB)],
                              idx_v.at[ibuf], semi).wait()

    def issue(ibuf, srow, slot):
        pltpu.async_copy(table_hbm.at[idx_v.at[ibuf, srow]],
                         gath_v.at[slot], sems[slot])

    def drain(slot):
        pltpu.make_async_copy(table_hbm.at[pl.ds(0, IDXW)],
                              gath_v.at[slot], sems[slot]).wait()

    def accum_store(k, slot_o):
        z = jnp.zeros((LANES,), jnp.float32)
        accs = (z, z, z, z)
        for (s, off, ln) in SEGS[k]:
            slot = s % NRING

            def body(i, a, slot=slot, off=off):
                a0, a1, a2, a3 = a
                a0 = a0 + gath_v[slot, off + i, pl.ds(0, LANES)]
                a1 = a1 + gath_v[slot, off + i, pl.ds(LANES, LANES)]
                a2 = a2 + gath_v[slot, off + i, pl.ds(2 * LANES, LANES)]
                a3 = a3 + gath_v[slot, off + i, pl.ds(3 * LANES, LANES)]
                return (a0, a1, a2, a3)

            accs = plsc.parallel_loop(0, ln, unroll=4, carry=accs)(body)
        scale = jnp.float32(1.0 / L)
        a0, a1, a2, a3 = accs
        outb_v[slot_o, pl.ds(0, LANES)] = a0 * scale
        outb_v[slot_o, pl.ds(LANES, LANES)] = a1 * scale
        outb_v[slot_o, pl.ds(2 * LANES, LANES)] = a2 * scale
        outb_v[slot_o, pl.ds(3 * LANES, LANES)] = a3 * scale

    stage_idx(0, 0)
    for g in range(NRING):
        issue(0, g, g)

    @pl.loop(0, NBLK // 2)
    def _blocks(m2):
        for half in range(2):
            ib = half
            m = 2 * m2 + half

            @pl.when(m < NBLK - 1)
            def _(ib=ib, m=m):
                stage_idx(m + 1, 1 - ib)

            for k in range(RPB):
                for s in NEW_STREAMS[k]:
                    drain(s % NRING)
                slot_o = k + RPB * half + 2 * RPB * (m2 % 2)
                accum_store(k, slot_o)
                for s in DEAD_STREAMS[k]:
                    if s + NRING < SPB:
                        issue(ib, s + NRING, s % NRING)
                    else:
                        @pl.when(m < NBLK - 1)
                        def _(ib=ib, s=s):
                            issue(1 - ib, s + NRING - SPB, s % NRING)

            if half == 1:
                @pl.when(m2 % 2 == 1)
                def _(m2=m2):
                    pltpu.sync_copy(
                        outb_v,
                        out_hbm.at[pl.ds(wid * ROWS_PER_W
                                         + (m2 // 2) * OCHUNK, OCHUNK)])


def kernel(x, table):
    x2 = x.astype(jnp.int32).reshape(NW * STREAMS_PER_W, IDXW)
    mesh = plsc.VectorSubcoreMesh(core_axis_name="c", subcore_axis_name="s")
    f = pl.kernel(
        _awe_body,
        out_type=jax.ShapeDtypeStruct((B, E), jnp.float32),
        mesh=mesh,
        scratch_types=[
            pltpu.VMEM((2, SPB, IDXW), jnp.int32),
            pltpu.VMEM((NRING, IDXW, E), jnp.float32),
            pltpu.VMEM((OCHUNK, E), jnp.float32),
            pltpu.SemaphoreType.DMA,
            pltpu.SemaphoreType.DMA,
            pltpu.SemaphoreType.DMA,
            pltpu.SemaphoreType.DMA,
            pltpu.SemaphoreType.DMA,
            pltpu.SemaphoreType.DMA,
        ],
        compiler_params=pltpu.CompilerParams(use_tc_tiling_on_sc=False),
    )
    return f(x2, table)


# speedup vs baseline: 1.1015x; 1.1015x over previous
import numpy as np
import jax
import jax.numpy as jnp
from jax import lax
from jax.experimental import pallas as pl
from jax.experimental.pallas import tpu as pltpu
from jax.experimental.pallas import tpu_sc as plsc

B = 16384
L = 200
E = 64
NC = 2
NS = 16
NW = NC * NS
ROWS_PER_W = B // NW
HALF = L // 2
RCHUNK = 128
NCHUNK = ROWS_PER_W // RCHUNK
OCHUNK = 128
LANES = 16

NBUF = 4


def _awe_body(x_hbm, table_hbm, out_hbm, idx_v, gath_v, outb_v,
              sem0, sem1, sem2, sem3, semi):
    wid = lax.axis_index("s") * NC + lax.axis_index("c")
    sems = (sem0, sem1, sem2, sem3)

    def issue_idx(c, ib):
        pltpu.async_copy(
            x_hbm.at[pl.ds(wid * (2 * ROWS_PER_W) + c * (2 * RCHUNK),
                           2 * RCHUNK)],
            idx_v.at[ib], semi)

    def drain_idx(ib):
        pltpu.make_async_copy(x_hbm.at[pl.ds(0, 2 * RCHUNK)],
                              idx_v.at[ib], semi).wait()

    def issue(ib, rloc, b):
        pltpu.async_copy(table_hbm.at[idx_v.at[ib, 2 * rloc]],
                         gath_v.at[b, pl.ds(0, HALF)], sems[b])
        pltpu.async_copy(table_hbm.at[idx_v.at[ib, 2 * rloc + 1]],
                         gath_v.at[b, pl.ds(HALF, HALF)], sems[b])

    def drain(b):
        pltpu.make_async_copy(table_hbm.at[pl.ds(0, L)],
                              gath_v.at[b], sems[b]).wait()

    def accum_store(rloc, b):
        def body(i, accs):
            a0, a1, a2, a3 = accs
            v0 = gath_v[b, i, pl.ds(0, LANES)]
            v1 = gath_v[b, i, pl.ds(LANES, LANES)]
            v2 = gath_v[b, i, pl.ds(2 * LANES, LANES)]
            v3 = gath_v[b, i, pl.ds(3 * LANES, LANES)]
            return (a0 + v0, a1 + v1, a2 + v2, a3 + v3)

        z = jnp.zeros((LANES,), jnp.float32)
        a0, a1, a2, a3 = plsc.parallel_loop(0, L, unroll=8,
                                            carry=(z, z, z, z))(body)
        scale = jnp.float32(1.0 / L)
        slot = rloc % OCHUNK
        outb_v[slot, pl.ds(0, LANES)] = a0 * scale
        outb_v[slot, pl.ds(LANES, LANES)] = a1 * scale
        outb_v[slot, pl.ds(2 * LANES, LANES)] = a2 * scale
        outb_v[slot, pl.ds(3 * LANES, LANES)] = a3 * scale

    issue_idx(0, 0)
    drain_idx(0)
    for c in range(NCHUNK):
        ib = c % 2
        if c + 1 < NCHUNK:
            issue_idx(c + 1, 1 - ib)
            drain_idx(1 - ib)

        if c == 0:
            for j in range(NBUF - 1):
                issue(ib, j, j)

        @pl.loop(0, RCHUNK, step=NBUF)
        def _rows(r):
            for j in range(NBUF):
                t = r + j + NBUF - 1
                buf = (j + NBUF - 1) % NBUF

                @pl.when(t < RCHUNK)
                def _():
                    issue(ib, t, buf)

                if c + 1 < NCHUNK:
                    @pl.when(t >= RCHUNK)
                    def _():
                        issue(1 - ib, t - RCHUNK, buf)

                drain(j)
                accum_store(r + j, j)

            @pl.when(r % OCHUNK == OCHUNK - NBUF)
            def _():
                pltpu.sync_copy(
                    outb_v,
                    out_hbm.at[pl.ds(wid * ROWS_PER_W + c * RCHUNK
                                     + (r // OCHUNK) * OCHUNK, OCHUNK)])


def kernel(x, table):
    x2 = x.astype(jnp.int32).reshape(2 * B, HALF)
    mesh = plsc.VectorSubcoreMesh(core_axis_name="c", subcore_axis_name="s")
    f = pl.kernel(
        _awe_body,
        out_type=jax.ShapeDtypeStruct((B, E), jnp.float32),
        mesh=mesh,
        scratch_types=[
            pltpu.VMEM((2, 2 * RCHUNK, HALF), jnp.int32),
            pltpu.VMEM((NBUF, L, E), jnp.float32),
            pltpu.VMEM((OCHUNK, E), jnp.float32),
            pltpu.SemaphoreType.DMA,
            pltpu.SemaphoreType.DMA,
            pltpu.SemaphoreType.DMA,
            pltpu.SemaphoreType.DMA,
            pltpu.SemaphoreType.DMA,
        ],
        compiler_params=pltpu.CompilerParams(use_tc_tiling_on_sc=False),
    )
    return f(x2, table)
